# Initial kernel scaffold; baseline (speedup 1.0000x reference)
import jax
import jax.numpy as jnp
from jax.experimental import pallas as pl


def kernel(x, graph, edge_index):
    raise NotImplementedError("write your pallas kernel here")


# speedup vs baseline: 1.3790x; 1.3790x over previous
import jax
import jax.numpy as jnp
from jax.experimental import pallas as pl


def _heads_body(x_ref, cons_ref, obj_ref):
    xb = x_ref[:]
    m = jnp.max(xb, axis=1, keepdims=True)
    e = jnp.exp(xb - m)
    s = jnp.sum(e, axis=1, keepdims=True)
    cons_ref[:] = e / s
    obj_ref[:] = jax.nn.sigmoid(xb.T)


def kernel(x, graph, edge_index):
    del graph, edge_index
    n, d = x.shape
    cons, obj = pl.pallas_call(
        _heads_body,
        out_shape=[
            jax.ShapeDtypeStruct((n, d), x.dtype),
            jax.ShapeDtypeStruct((d, n), x.dtype),
        ],
    )(x)
    return (cons, obj)
